# baseline (speedup 1.0000x reference)
import jax
import jax.numpy as jnp
from jax import lax
from jax.experimental import pallas as pl
from jax.experimental.pallas import tpu as pltpu
from jax.experimental.pallas import tpu_sc as plsc

_B, _L, _V, _D = 1024, 200, 100000, 128
_NW = 32
_SEQ_PER_W = _B // _NW
_HALF = 100
_LANES = 16
_NBUF = 4
_GRP = 4
_NGRP = _SEQ_PER_W // _GRP


def _grp_rows(g):
    return 2 * min(_GRP, _SEQ_PER_W - _GRP * g)


def _emb_body(idx_hbm, tok_hbm, pos_hbm, out_hbm, idx0, idx1, pos_v,
              buf0, buf1, buf2, buf3, isem, g0, g1, g2, g3, s0, s1, s2, s3):
    wid = lax.axis_index("s") * 2 + lax.axis_index("c")
    idx_s = (idx0, idx1)
    bufs = (buf0, buf1, buf2, buf3)
    gsems = (g0, g1, g2, g3)
    ssems = (s0, s1, s2, s3)

    def stage_desc(g):
        n = _grp_rows(g)
        return (idx_hbm.at[pl.ds(64 * wid + 2 * _GRP * g, n)],
                idx_s[g % 2].at[pl.ds(0, n)], isem)

    pltpu.sync_copy(pos_hbm, pos_v)
    pltpu.sync_copy(*stage_desc(0)[:2])

    def gather_descs(c):
        b = c % _NBUF
        slot = idx_s[(c // _GRP) % 2]
        r = 2 * (c % _GRP)
        return (
            (tok_hbm.at[slot.at[r]], bufs[b].at[pl.ds(0, _HALF)], gsems[b]),
            (tok_hbm.at[slot.at[r + 1]], bufs[b].at[pl.ds(_HALF, _HALF)],
             gsems[b]),
        )

    def scatter_desc(c):
        b = c % _NBUF
        return (bufs[b], out_hbm.at[_SEQ_PER_W * wid + c], ssems[b])

    def issue_gather(c):
        for d in gather_descs(c):
            pltpu.async_copy(*d)

    def wait_gather(c):
        for d in gather_descs(c):
            pltpu.make_async_copy(*d).wait()

    def add_pos(buf):
        def row(r, rc):
            for u in range(4):
                for j in range(_D // _LANES):
                    sl = pl.ds(_LANES * j, _LANES)
                    plsc.addupdate(buf.at[4 * r + u, sl], pos_v[4 * r + u, sl])
            return rc
        lax.fori_loop(0, _L // 4, row, 0)

    issue_gather(0)
    issue_gather(1)
    for c in range(_SEQ_PER_W):
        b = c % _NBUF
        wait_gather(c)
        if c % _GRP == 0 and c // _GRP + 1 < _NGRP:
            pltpu.async_copy(*stage_desc(c // _GRP + 1))
        if c + 2 < _SEQ_PER_W:
            if c >= 2:
                pltpu.make_async_copy(*scatter_desc(c - 2)).wait()
            if (c + 2) % _GRP == 0:
                pltpu.make_async_copy(*stage_desc((c + 2) // _GRP)).wait()
            issue_gather(c + 2)
        add_pos(bufs[b])
        pltpu.async_copy(*scatter_desc(c))
    for c in range(_SEQ_PER_W - _NBUF, _SEQ_PER_W):
        pltpu.make_async_copy(*scatter_desc(c)).wait()


def kernel(x, token_table, pos_table):
    idx2 = x.astype(jnp.int32).reshape(_B * _L // _HALF, _HALF)
    mesh = plsc.VectorSubcoreMesh(core_axis_name="c", subcore_axis_name="s")
    run = pl.kernel(
        _emb_body,
        out_type=jax.ShapeDtypeStruct((_B, _L, _D), jnp.float32),
        mesh=mesh,
        scratch_types=(
            [pltpu.VMEM((2 * _GRP, _HALF), jnp.int32),
             pltpu.VMEM((2 * _GRP, _HALF), jnp.int32),
             pltpu.VMEM((_L, _D), jnp.float32)]
            + [pltpu.VMEM((_L, _D), jnp.float32) for _ in range(_NBUF)]
            + [pltpu.SemaphoreType.DMA for _ in range(2 * _NBUF + 1)]
        ),
    )
    return run(idx2, token_table, pos_table)


# speedup vs baseline: 1.0067x; 1.0067x over previous
import jax
import jax.numpy as jnp
from jax import lax
from jax.experimental import pallas as pl
from jax.experimental.pallas import tpu as pltpu
from jax.experimental.pallas import tpu_sc as plsc

_B, _L, _V, _D = 1024, 200, 100000, 128
_NW = 32
_SEQ_PER_W = _B // _NW
_HALF = 100
_LANES = 16
_NBUF = 4
_GRP = 4
_NGRP = _SEQ_PER_W // _GRP


def _grp_rows(g):
    return 2 * min(_GRP, _SEQ_PER_W - _GRP * g)


def _emb_body(idx_hbm, tok_hbm, pos_hbm, out_hbm, idx0, idx1, pos_v,
              buf0, buf1, buf2, buf3, isem, g0, g1, g2, g3, s0, s1, s2, s3):
    wid = lax.axis_index("s") * 2 + lax.axis_index("c")
    idx_s = (idx0, idx1)
    bufs = (buf0, buf1, buf2, buf3)
    gsems = (g0, g1, g2, g3)
    ssems = (s0, s1, s2, s3)

    def stage_desc(g):
        n = _grp_rows(g)
        return (idx_hbm.at[pl.ds(64 * wid + 2 * _GRP * g, n)],
                idx_s[g % 2].at[pl.ds(0, n)], isem)

    pltpu.sync_copy(pos_hbm, pos_v)
    pltpu.sync_copy(*stage_desc(0)[:2])

    def gather_descs(c):
        b = c % _NBUF
        slot = idx_s[(c // _GRP) % 2]
        r = 2 * (c % _GRP)
        return (
            (tok_hbm.at[slot.at[r]], bufs[b].at[pl.ds(0, _HALF)], gsems[b]),
            (tok_hbm.at[slot.at[r + 1]], bufs[b].at[pl.ds(_HALF, _HALF)],
             gsems[b]),
        )

    def scatter_descs(c):
        b = c % _NBUF
        seq = out_hbm.at[_SEQ_PER_W * wid + c]
        return (
            (bufs[b].at[pl.ds(0, 96)], seq.at[pl.ds(0, 96)], ssems[b]),
            (bufs[b].at[pl.ds(96, 104)], seq.at[pl.ds(96, 104)], ssems[b]),
        )

    def issue_gather(c):
        for d in gather_descs(c):
            pltpu.async_copy(*d)

    def wait_gather(c):
        for d in gather_descs(c):
            pltpu.make_async_copy(*d).wait()

    def add_pos(buf, r0, nrows):
        def row(r, rc):
            for u in range(2):
                for j in range(_D // _LANES):
                    sl = pl.ds(_LANES * j, _LANES)
                    plsc.addupdate(buf.at[r0 + 2 * r + u, sl],
                                   pos_v[r0 + 2 * r + u, sl])
            return rc
        lax.fori_loop(0, nrows // 2, row, 0)

    issue_gather(0)
    issue_gather(1)
    for c in range(_SEQ_PER_W):
        b = c % _NBUF
        wait_gather(c)
        if c % _GRP == 0 and c // _GRP + 1 < _NGRP:
            pltpu.async_copy(*stage_desc(c // _GRP + 1))
        if c + 2 < _SEQ_PER_W:
            if c >= 2:
                for d in scatter_descs(c - 2):
                    pltpu.make_async_copy(*d).wait()
            if (c + 2) % _GRP == 0:
                pltpu.make_async_copy(*stage_desc((c + 2) // _GRP)).wait()
            issue_gather(c + 2)
        add_pos(bufs[b], 0, 96)
        pltpu.async_copy(*scatter_descs(c)[0])
        add_pos(bufs[b], 96, 104)
        pltpu.async_copy(*scatter_descs(c)[1])
    for c in range(_SEQ_PER_W - _NBUF, _SEQ_PER_W):
        for d in scatter_descs(c):
            pltpu.make_async_copy(*d).wait()


def kernel(x, token_table, pos_table):
    idx2 = x.astype(jnp.int32).reshape(_B * _L // _HALF, _HALF)
    mesh = plsc.VectorSubcoreMesh(core_axis_name="c", subcore_axis_name="s")
    run = pl.kernel(
        _emb_body,
        out_type=jax.ShapeDtypeStruct((_B, _L, _D), jnp.float32),
        mesh=mesh,
        scratch_types=(
            [pltpu.VMEM((2 * _GRP, _HALF), jnp.int32),
             pltpu.VMEM((2 * _GRP, _HALF), jnp.int32),
             pltpu.VMEM((_L, _D), jnp.float32)]
            + [pltpu.VMEM((_L, _D), jnp.float32) for _ in range(_NBUF)]
            + [pltpu.SemaphoreType.DMA for _ in range(2 * _NBUF + 1)]
        ),
    )
    return run(idx2, token_table, pos_table)


# speedup vs baseline: 1.0403x; 1.0334x over previous
import jax
import jax.numpy as jnp
from jax import lax
from jax.experimental import pallas as pl
from jax.experimental.pallas import tpu as pltpu
from jax.experimental.pallas import tpu_sc as plsc

_B, _L, _V, _D = 1024, 200, 100000, 128
_NW = 32
_SEQ_PER_W = _B // _NW
_HALF = 100
_LANES = 16
_NBUF = 4
_GRP = 4
_NGRP = _SEQ_PER_W // _GRP


def _grp_rows(g):
    return 2 * min(_GRP, _SEQ_PER_W - _GRP * g)


def _emb_body(idx_hbm, tok_hbm, pos_hbm, out_hbm, idx0, idx1, pos_v,
              buf0, buf1, buf2, buf3, isem, g0, g1, g2, g3, s0, s1, s2, s3):
    wid = lax.axis_index("s") * 2 + lax.axis_index("c")
    idx_s = (idx0, idx1)
    bufs = (buf0, buf1, buf2, buf3)
    gsems = (g0, g1, g2, g3)
    ssems = (s0, s1, s2, s3)

    def stage_desc(g):
        n = _grp_rows(g)
        return (idx_hbm.at[pl.ds(64 * wid + 2 * _GRP * g, n)],
                idx_s[g % 2].at[pl.ds(0, n)], isem)

    pltpu.sync_copy(pos_hbm, pos_v)
    pltpu.sync_copy(*stage_desc(0)[:2])

    def gather_descs(c):
        b = c % _NBUF
        slot = idx_s[(c // _GRP) % 2]
        r = 2 * (c % _GRP)
        return (
            (tok_hbm.at[slot.at[r]], bufs[b].at[pl.ds(0, _HALF)], gsems[b]),
            (tok_hbm.at[slot.at[r + 1]], bufs[b].at[pl.ds(_HALF, _HALF)],
             gsems[b]),
        )

    def scatter_desc(c):
        b = c % _NBUF
        return (bufs[b], out_hbm.at[_SEQ_PER_W * wid + c], ssems[b])

    def issue_gather(c):
        for d in gather_descs(c):
            pltpu.async_copy(*d)

    def wait_gather(c):
        for d in gather_descs(c):
            pltpu.make_async_copy(*d).wait()

    def add_pos(buf):
        def row(r, rc):
            for u in range(2):
                for j in range(_D // _LANES):
                    sl = pl.ds(_LANES * j, _LANES)
                    plsc.addupdate(buf.at[2 * r + u, sl], pos_v[2 * r + u, sl])
            return rc
        lax.fori_loop(0, _L // 2, row, 0)

    issue_gather(0)
    issue_gather(1)
    for c in range(_SEQ_PER_W):
        b = c % _NBUF
        wait_gather(c)
        if c % _GRP == 0 and c // _GRP + 1 < _NGRP:
            pltpu.async_copy(*stage_desc(c // _GRP + 1))
        if c + 2 < _SEQ_PER_W:
            if c >= 2:
                pltpu.make_async_copy(*scatter_desc(c - 2)).wait()
            if (c + 2) % _GRP == 0:
                pltpu.make_async_copy(*stage_desc((c + 2) // _GRP)).wait()
            issue_gather(c + 2)
        add_pos(bufs[b])
        pltpu.async_copy(*scatter_desc(c))
    for c in range(_SEQ_PER_W - _NBUF, _SEQ_PER_W):
        pltpu.make_async_copy(*scatter_desc(c)).wait()


def kernel(x, token_table, pos_table):
    idx2 = x.astype(jnp.int32).reshape(_B * _L // _HALF, _HALF)
    mesh = plsc.VectorSubcoreMesh(core_axis_name="c", subcore_axis_name="s")
    run = pl.kernel(
        _emb_body,
        out_type=jax.ShapeDtypeStruct((_B, _L, _D), jnp.float32),
        mesh=mesh,
        scratch_types=(
            [pltpu.VMEM((2 * _GRP, _HALF), jnp.int32),
             pltpu.VMEM((2 * _GRP, _HALF), jnp.int32),
             pltpu.VMEM((_L, _D), jnp.float32)]
            + [pltpu.VMEM((_L, _D), jnp.float32) for _ in range(_NBUF)]
            + [pltpu.SemaphoreType.DMA for _ in range(2 * _NBUF + 1)]
        ),
    )
    return run(idx2, token_table, pos_table)
